# baseline (speedup 1.0000x reference)
import functools

import jax
import jax.numpy as jnp
from jax import lax
from jax.experimental import pallas as pl
from jax.experimental.pallas import tpu as pltpu
from jax.experimental.pallas import tpu_sc as plsc

_VOCAB = 100000
_EMBED_DIM = 128
_N_ROWS = 64
_STRIDE = 1562


def _sc_gather(x):
    mesh = plsc.ScalarSubcoreMesh(axis_name="c", num_cores=2)

    @functools.partial(
        pl.kernel,
        mesh=mesh,
        out_type=jax.ShapeDtypeStruct((_N_ROWS, _EMBED_DIM), jnp.float32),
        scratch_types=[
            pltpu.VMEM_SHARED((4, 8, _EMBED_DIM), jnp.float32),
            pltpu.SemaphoreType.DMA,
        ],
    )
    def k(x_hbm, out_hbm, sp, sem):
        cid = lax.axis_index("c")
        x3 = x_hbm.at[pl.ds(0, _N_ROWS * _STRIDE)].reshape(
            8, 8 * _STRIDE, _EMBED_DIM
        )
        for r in range(8):
            pltpu.async_copy(
                x3.at[pl.ds(cid * 4, 4), pl.ds(r * _STRIDE, 1), :],
                sp.at[:, pl.ds(r, 1), :],
                sem,
            )
        pltpu.make_async_copy(x_hbm.at[pl.ds(0, _N_ROWS // 2)], sp, sem).wait()
        pltpu.sync_copy(
            sp.reshape(_N_ROWS // 2, _EMBED_DIM),
            out_hbm.at[pl.ds(cid * (_N_ROWS // 2), _N_ROWS // 2)],
        )

    return k(x)


def kernel(x):
    return _sc_gather(x)


# speedup vs baseline: 1.0804x; 1.0804x over previous
import functools

import jax
import jax.numpy as jnp
from jax.experimental import pallas as pl
from jax.experimental.pallas import tpu as pltpu
from jax.experimental.pallas import tpu_sc as plsc

_VOCAB = 100000
_EMBED_DIM = 128
_N_ROWS = 64
_STRIDE = 1562


def _sc_gather(x):
    mesh = plsc.ScalarSubcoreMesh(axis_name="c", num_cores=1)

    @functools.partial(
        pl.kernel,
        mesh=mesh,
        out_type=jax.ShapeDtypeStruct((_N_ROWS, _EMBED_DIM), jnp.float32),
        scratch_types=[
            pltpu.VMEM_SHARED((8, 8, _EMBED_DIM), jnp.float32),
            pltpu.SemaphoreType.DMA,
        ],
    )
    def k(x_hbm, out_hbm, sp, sem):
        x3 = x_hbm.at[pl.ds(0, _N_ROWS * _STRIDE)].reshape(
            8, 8 * _STRIDE, _EMBED_DIM
        )
        copies = [
            pltpu.async_copy(
                x3.at[:, pl.ds(r * _STRIDE, 1), :],
                sp.at[:, pl.ds(r, 1), :],
                sem,
            )
            for r in range(8)
        ]
        for c in copies:
            c.wait()
        pltpu.sync_copy(sp.reshape(_N_ROWS, _EMBED_DIM), out_hbm)

    return k(x)


def kernel(x):
    return _sc_gather(x)
